# baseline (speedup 1.0000x reference)
import functools

import jax
import jax.numpy as jnp
from jax import lax
from jax.experimental import pallas as pl
from jax.experimental.pallas import tpu as pltpu
from jax.experimental.pallas import tpu_sc as plsc

B = 320000
D = 128
C = 128
L = 16
NW = 32
BW = B // NW
NFULL = BW // C
TAIL = BW - NFULL * C
NBUF = 2


@jax.jit
def _impl(x_user, x_movie, u_idx, m_idx):
    mesh = plsc.VectorSubcoreMesh(core_axis_name="c", subcore_axis_name="s")

    @functools.partial(
        pl.kernel,
        mesh=mesh,
        out_type=jax.ShapeDtypeStruct((B,), jnp.float32),
        scratch_types=[
            pltpu.VMEM((BW,), jnp.int32),
            pltpu.VMEM((BW,), jnp.int32),
            pltpu.VMEM((C, D), jnp.bfloat16),
            pltpu.VMEM((C, D), jnp.bfloat16),
            pltpu.VMEM((C, D), jnp.bfloat16),
            pltpu.VMEM((C, D), jnp.bfloat16),
            pltpu.VMEM((BW,), jnp.float32),
            pltpu.VMEM((L * L,), jnp.float32),
            pltpu.SemaphoreType.DMA,
            pltpu.SemaphoreType.DMA,
            pltpu.SemaphoreType.DMA,
            pltpu.SemaphoreType.DMA,
        ],
        compiler_params=pltpu.CompilerParams(
            needs_layout_passes=False,
            disable_bounds_checks=True,
            use_tc_tiling_on_sc=False,
        ),
    )
    def k(u_hbm, m_hbm, uidx_hbm, midx_hbm, out_hbm,
          uidx_v, midx_v, u0, m0, u1, m1, out_v, tr_v,
          su0, sm0, su1, sm1):
        ubuf = (u0, u1)
        mbuf = (m0, m1)
        usem = (su0, su1)
        msem = (sm0, sm1)
        wid = lax.axis_index("c") * 16 + lax.axis_index("s")
        wbase = wid * BW
        pltpu.sync_copy(uidx_hbm.at[pl.ds(wbase, BW)], uidx_v)
        pltpu.sync_copy(midx_hbm.at[pl.ds(wbase, BW)], midx_v)

        lane = lax.iota(jnp.int32, L)
        lane16 = lane * L

        def fire(i, b):
            pltpu.async_copy(
                u_hbm.at[uidx_v.at[pl.ds(i * C, C)]], ubuf[b], usem[b])
            pltpu.async_copy(
                m_hbm.at[midx_v.at[pl.ds(i * C, C)]], mbuf[b], msem[b])

        def drain(i, b):
            pltpu.make_async_copy(
                u_hbm.at[uidx_v.at[pl.ds(i * C, C)]], ubuf[b], usem[b]).wait()
            pltpu.make_async_copy(
                m_hbm.at[midx_v.at[pl.ds(i * C, C)]], mbuf[b], msem[b]).wait()

        def compute(i, ub, mb, n_rows):
            def group_body(g, _):
                for r in range(L):
                    row = g * L + r
                    terms = []
                    for t in range(D // (2 * L)):
                        au = ub[row, pl.ds(t * 2 * L, 2 * L)]
                        am = mb[row, pl.ds(t * 2 * L, 2 * L)]
                        ue, uo = plsc.unpack(
                            au, format=plsc.PackFormat.INTERLEAVED)
                        me, mo = plsc.unpack(
                            am, format=plsc.PackFormat.INTERLEAVED)
                        terms.append(ue * me)
                        terms.append(uo * mo)
                    s4 = [terms[2 * t] + terms[2 * t + 1] for t in range(4)]
                    s2 = [s4[0] + s4[1], s4[2] + s4[3]]
                    plsc.store_scatter(tr_v, [lane16 + r], s2[0] + s2[1])
                cols = [tr_v[pl.ds(c * L, L)] for c in range(L)]
                c8 = [cols[2 * t] + cols[2 * t + 1] for t in range(8)]
                c4 = [c8[2 * t] + c8[2 * t + 1] for t in range(4)]
                c2 = [c4[0] + c4[1], c4[2] + c4[3]]
                out_v[pl.ds(i * C + g * L, L)] = c2[0] + c2[1]
                return 0

            lax.fori_loop(0, n_rows // L, group_body, 0)

        for b in range(NBUF):
            fire(b, b)

        def ring_body(kk, _):
            base = NBUF * kk
            for b in range(NBUF):
                i = base + b
                drain(i, b)
                compute(i, ubuf[b], mbuf[b], C)
                fire(i + NBUF, b)
            return 0

        n_main = (NFULL - NBUF) // NBUF
        lax.fori_loop(0, n_main, ring_body, 0)

        i0 = n_main * NBUF
        drain(i0, 0)
        compute(i0, u0, m0, C)
        ut = u0.at[pl.ds(0, TAIL)]
        mt = m0.at[pl.ds(0, TAIL)]
        tidx_u = uidx_v.at[pl.ds(NFULL * C, TAIL)]
        tidx_m = midx_v.at[pl.ds(NFULL * C, TAIL)]
        pltpu.async_copy(u_hbm.at[tidx_u], ut, su0)
        pltpu.async_copy(m_hbm.at[tidx_m], mt, sm0)
        drain(i0 + 1, 1)
        compute(i0 + 1, u1, m1, C)
        pltpu.make_async_copy(u_hbm.at[tidx_u], ut, su0).wait()
        pltpu.make_async_copy(m_hbm.at[tidx_m], mt, sm0).wait()
        compute(NFULL, u0, m0, TAIL)

        pltpu.sync_copy(out_v, out_hbm.at[pl.ds(wbase, BW)])

    return k(x_user, x_movie, u_idx, m_idx)


def kernel(x_user, x_movie, edge_label_index):
    idx = edge_label_index.astype(jnp.int32)
    return _impl(x_user.astype(jnp.bfloat16), x_movie.astype(jnp.bfloat16),
                 idx[0], idx[1])


# speedup vs baseline: 1.2000x; 1.2000x over previous
import functools

import jax
import jax.numpy as jnp
from jax import lax
from jax.experimental import pallas as pl
from jax.experimental.pallas import tpu as pltpu
from jax.experimental.pallas import tpu_sc as plsc

B = 320000
D = 128
C = 128
L = 16
NW = 32
BW = B // NW
NFULL = BW // C
TAIL = BW - NFULL * C
NBUF = 2


@jax.jit
def _impl(x_user, x_movie, u_idx, m_idx):
    mesh = plsc.VectorSubcoreMesh(core_axis_name="c", subcore_axis_name="s")

    @functools.partial(
        pl.kernel,
        mesh=mesh,
        out_type=jax.ShapeDtypeStruct((B,), jnp.float32),
        scratch_types=[
            pltpu.VMEM((BW,), jnp.int32),
            pltpu.VMEM((BW,), jnp.int32),
            pltpu.VMEM((C, D), jnp.float32),
            pltpu.VMEM((C, D), jnp.float32),
            pltpu.VMEM((C, D), jnp.float32),
            pltpu.VMEM((C, D), jnp.float32),
            pltpu.VMEM((BW,), jnp.float32),
            pltpu.VMEM((2 * L * L,), jnp.float32),
            pltpu.SemaphoreType.DMA,
            pltpu.SemaphoreType.DMA,
            pltpu.SemaphoreType.DMA,
            pltpu.SemaphoreType.DMA,
        ],
        compiler_params=pltpu.CompilerParams(
            needs_layout_passes=False,
            disable_bounds_checks=True,
        ),
    )
    def k(u_hbm, m_hbm, uidx_hbm, midx_hbm, out_hbm,
          uidx_v, midx_v, u0, m0, u1, m1, out_v, tr_v,
          su0, sm0, su1, sm1):
        ubuf = (u0, u1)
        mbuf = (m0, m1)
        usem = (su0, su1)
        msem = (sm0, sm1)
        wid = lax.axis_index("c") * 16 + lax.axis_index("s")
        wbase = wid * BW
        pltpu.sync_copy(uidx_hbm.at[pl.ds(wbase, BW)], uidx_v)
        pltpu.sync_copy(midx_hbm.at[pl.ds(wbase, BW)], midx_v)

        lane = lax.iota(jnp.int32, L)
        lane16 = lane * L

        def fire(i, b):
            pltpu.async_copy(
                u_hbm.at[uidx_v.at[pl.ds(i * C, C)]], ubuf[b], usem[b])
            pltpu.async_copy(
                m_hbm.at[midx_v.at[pl.ds(i * C, C)]], mbuf[b], msem[b])

        def drain(i, b):
            pltpu.make_async_copy(
                u_hbm.at[uidx_v.at[pl.ds(i * C, C)]], ubuf[b], usem[b]).wait()
            pltpu.make_async_copy(
                m_hbm.at[midx_v.at[pl.ds(i * C, C)]], mbuf[b], msem[b]).wait()

        def compute(i, ub, mb, n_rows):
            def one_group(g, half):
                off = half * L * L
                for r in range(L):
                    row = g * L + r
                    terms = [ub[row, pl.ds(t * L, L)] * mb[row, pl.ds(t * L, L)]
                             for t in range(D // L)]
                    s4 = [terms[2 * t] + terms[2 * t + 1] for t in range(4)]
                    s2 = [s4[0] + s4[1], s4[2] + s4[3]]
                    plsc.store_scatter(tr_v, [lane16 + (off + r)],
                                       s2[0] + s2[1])
                cols = [tr_v[pl.ds(off + c * L, L)] for c in range(L)]
                c8 = [cols[2 * t] + cols[2 * t + 1] for t in range(8)]
                c4 = [c8[2 * t] + c8[2 * t + 1] for t in range(4)]
                c2 = [c4[0] + c4[1], c4[2] + c4[3]]
                out_v[pl.ds(i * C + g * L, L)] = c2[0] + c2[1]

            if n_rows >= 2 * L:
                def group_body(g2, _):
                    one_group(2 * g2, 0)
                    one_group(2 * g2 + 1, 1)
                    return 0

                lax.fori_loop(0, n_rows // (2 * L), group_body, 0)
            else:
                one_group(0, 0)

        for b in range(NBUF):
            fire(b, b)

        def ring_body(kk, _):
            base = NBUF * kk
            for b in range(NBUF):
                i = base + b
                drain(i, b)
                compute(i, ubuf[b], mbuf[b], C)
                fire(i + NBUF, b)
            return 0

        n_main = (NFULL - NBUF) // NBUF
        lax.fori_loop(0, n_main, ring_body, 0)

        i0 = n_main * NBUF
        drain(i0, 0)
        compute(i0, u0, m0, C)
        ut = u0.at[pl.ds(0, TAIL)]
        mt = m0.at[pl.ds(0, TAIL)]
        tidx_u = uidx_v.at[pl.ds(NFULL * C, TAIL)]
        tidx_m = midx_v.at[pl.ds(NFULL * C, TAIL)]
        pltpu.async_copy(u_hbm.at[tidx_u], ut, su0)
        pltpu.async_copy(m_hbm.at[tidx_m], mt, sm0)
        drain(i0 + 1, 1)
        compute(i0 + 1, u1, m1, C)
        pltpu.make_async_copy(u_hbm.at[tidx_u], ut, su0).wait()
        pltpu.make_async_copy(m_hbm.at[tidx_m], mt, sm0).wait()
        compute(NFULL, u0, m0, TAIL)

        pltpu.sync_copy(out_v, out_hbm.at[pl.ds(wbase, BW)])

    return k(x_user, x_movie, u_idx, m_idx)


def kernel(x_user, x_movie, edge_label_index):
    idx = edge_label_index.astype(jnp.int32)
    return _impl(x_user, x_movie, idx[0], idx[1])


# speedup vs baseline: 1.3895x; 1.1579x over previous
import functools

import jax
import jax.numpy as jnp
from jax import lax
from jax.experimental import pallas as pl
from jax.experimental.pallas import tpu as pltpu
from jax.experimental.pallas import tpu_sc as plsc

B = 320000
D = 128
C = 128
L = 16
NW = 32
BW = B // NW
NFULL = BW // C
TAIL = BW - NFULL * C
NBUF = 2


@jax.jit
def _impl(x_user, x_movie, u_idx, m_idx):
    mesh = plsc.VectorSubcoreMesh(core_axis_name="c", subcore_axis_name="s")

    @functools.partial(
        pl.kernel,
        mesh=mesh,
        out_type=jax.ShapeDtypeStruct((B,), jnp.float32),
        scratch_types=[
            pltpu.VMEM((BW,), jnp.int32),
            pltpu.VMEM((BW,), jnp.int32),
            pltpu.VMEM((C, D), jnp.float32),
            pltpu.VMEM((C, D), jnp.float32),
            pltpu.VMEM((C, D), jnp.float32),
            pltpu.VMEM((C, D), jnp.float32),
            pltpu.VMEM((BW,), jnp.float32),
            pltpu.VMEM((8 * L * L,), jnp.float32),
            pltpu.SemaphoreType.DMA,
            pltpu.SemaphoreType.DMA,
            pltpu.SemaphoreType.DMA,
            pltpu.SemaphoreType.DMA,
        ],
        compiler_params=pltpu.CompilerParams(
            needs_layout_passes=False,
            disable_bounds_checks=True,
        ),
    )
    def k(u_hbm, m_hbm, uidx_hbm, midx_hbm, out_hbm,
          uidx_v, midx_v, u0, m0, u1, m1, out_v, tr_v,
          su0, sm0, su1, sm1):
        ubuf = (u0, u1)
        mbuf = (m0, m1)
        usem = (su0, su1)
        msem = (sm0, sm1)
        wid = lax.axis_index("c") * 16 + lax.axis_index("s")
        wbase = wid * BW
        pltpu.sync_copy(uidx_hbm.at[pl.ds(wbase, BW)], uidx_v)
        pltpu.sync_copy(midx_hbm.at[pl.ds(wbase, BW)], midx_v)

        lane = lax.iota(jnp.int32, L)
        lane16 = lane * L

        def fire(i, b):
            pltpu.async_copy(
                u_hbm.at[uidx_v.at[pl.ds(i * C, C)]], ubuf[b], usem[b])
            pltpu.async_copy(
                m_hbm.at[midx_v.at[pl.ds(i * C, C)]], mbuf[b], msem[b])

        def drain(i, b):
            pltpu.make_async_copy(
                u_hbm.at[uidx_v.at[pl.ds(i * C, C)]], ubuf[b], usem[b]).wait()
            pltpu.make_async_copy(
                m_hbm.at[midx_v.at[pl.ds(i * C, C)]], mbuf[b], msem[b]).wait()

        def compute(i, ub, mb, n_rows):
            @plsc.parallel_loop(0, n_rows // L)
            def group_body(g):
                off = g * (L * L)
                for r in range(L):
                    row = g * L + r
                    terms = [ub[row, pl.ds(t * L, L)] * mb[row, pl.ds(t * L, L)]
                             for t in range(D // L)]
                    s4 = [terms[2 * t] + terms[2 * t + 1] for t in range(4)]
                    s2 = [s4[0] + s4[1], s4[2] + s4[3]]
                    plsc.store_scatter(tr_v, [lane16 + (off + r)],
                                       s2[0] + s2[1])
                cols = [tr_v[pl.ds(off + c * L, L)] for c in range(L)]
                c8 = [cols[2 * t] + cols[2 * t + 1] for t in range(8)]
                c4 = [c8[2 * t] + c8[2 * t + 1] for t in range(4)]
                c2 = [c4[0] + c4[1], c4[2] + c4[3]]
                out_v[pl.ds(i * C + g * L, L)] = c2[0] + c2[1]

        for b in range(NBUF):
            fire(b, b)

        def ring_body(kk, _):
            base = NBUF * kk
            for b in range(NBUF):
                i = base + b
                drain(i, b)
                compute(i, ubuf[b], mbuf[b], C)
                fire(i + NBUF, b)
            return 0

        n_main = (NFULL - NBUF) // NBUF
        lax.fori_loop(0, n_main, ring_body, 0)

        i0 = n_main * NBUF
        drain(i0, 0)
        compute(i0, u0, m0, C)
        ut = u0.at[pl.ds(0, TAIL)]
        mt = m0.at[pl.ds(0, TAIL)]
        tidx_u = uidx_v.at[pl.ds(NFULL * C, TAIL)]
        tidx_m = midx_v.at[pl.ds(NFULL * C, TAIL)]
        pltpu.async_copy(u_hbm.at[tidx_u], ut, su0)
        pltpu.async_copy(m_hbm.at[tidx_m], mt, sm0)
        drain(i0 + 1, 1)
        compute(i0 + 1, u1, m1, C)
        pltpu.make_async_copy(u_hbm.at[tidx_u], ut, su0).wait()
        pltpu.make_async_copy(m_hbm.at[tidx_m], mt, sm0).wait()
        compute(NFULL, u0, m0, TAIL)

        pltpu.sync_copy(out_v, out_hbm.at[pl.ds(wbase, BW)])

    return k(x_user, x_movie, u_idx, m_idx)


def kernel(x_user, x_movie, edge_label_index):
    idx = edge_label_index.astype(jnp.int32)
    return _impl(x_user, x_movie, idx[0], idx[1])


# speedup vs baseline: 1.4121x; 1.0162x over previous
import functools

import jax
import jax.numpy as jnp
from jax import lax
from jax.experimental import pallas as pl
from jax.experimental.pallas import tpu as pltpu
from jax.experimental.pallas import tpu_sc as plsc

B = 320000
D = 128
C = 128
L = 16
NW = 32
BW = B // NW
NFULL = BW // C
TAIL = BW - NFULL * C
NBUF = 2


@jax.jit
def _impl(x_user, x_movie, u_idx, m_idx):
    mesh = plsc.VectorSubcoreMesh(core_axis_name="c", subcore_axis_name="s")

    @functools.partial(
        pl.kernel,
        mesh=mesh,
        out_type=jax.ShapeDtypeStruct((B,), jnp.float32),
        scratch_types=[
            pltpu.VMEM((BW,), jnp.int32),
            pltpu.VMEM((BW,), jnp.int32),
            pltpu.VMEM((C, D), jnp.float32),
            pltpu.VMEM((C, D), jnp.float32),
            pltpu.VMEM((C, D), jnp.float32),
            pltpu.VMEM((C, D), jnp.float32),
            pltpu.VMEM((BW,), jnp.float32),
            pltpu.VMEM((8 * L * L,), jnp.float32),
            pltpu.SemaphoreType.DMA,
            pltpu.SemaphoreType.DMA,
            pltpu.SemaphoreType.DMA,
            pltpu.SemaphoreType.DMA,
        ],
        compiler_params=pltpu.CompilerParams(
            needs_layout_passes=False,
            disable_bounds_checks=True,
        ),
    )
    def k(u_hbm, m_hbm, uidx_hbm, midx_hbm, out_hbm,
          uidx_v, midx_v, u0, m0, u1, m1, out_v, tr_v,
          su0, sm0, su1, sm1):
        ubuf = (u0, u1)
        mbuf = (m0, m1)
        usem = (su0, su1)
        msem = (sm0, sm1)
        wid = lax.axis_index("c") * 16 + lax.axis_index("s")
        wbase = wid * BW
        pltpu.sync_copy(uidx_hbm.at[pl.ds(wbase, BW)], uidx_v)
        pltpu.sync_copy(midx_hbm.at[pl.ds(wbase, BW)], midx_v)

        lane = lax.iota(jnp.int32, L)
        lane16 = lane * L

        def fire(i, b):
            pltpu.async_copy(
                u_hbm.at[uidx_v.at[pl.ds(i * C, C)]], ubuf[b], usem[b])
            pltpu.async_copy(
                m_hbm.at[midx_v.at[pl.ds(i * C, C)]], mbuf[b], msem[b])

        def drain(i, b):
            pltpu.make_async_copy(
                u_hbm.at[uidx_v.at[pl.ds(i * C, C)]], ubuf[b], usem[b]).wait()
            pltpu.make_async_copy(
                m_hbm.at[midx_v.at[pl.ds(i * C, C)]], mbuf[b], msem[b]).wait()

        def compute(i, ub, mb, n_rows):
            @plsc.parallel_loop(0, n_rows // L, unroll=2 if n_rows > L else 1)
            def group_body(g):
                off = g * (L * L)
                for r in range(L):
                    row = g * L + r
                    terms = [ub[row, pl.ds(t * L, L)] * mb[row, pl.ds(t * L, L)]
                             for t in range(D // L)]
                    s4 = [terms[2 * t] + terms[2 * t + 1] for t in range(4)]
                    s2 = [s4[0] + s4[1], s4[2] + s4[3]]
                    plsc.store_scatter(tr_v, [lane16 + (off + r)],
                                       s2[0] + s2[1])
                cols = [tr_v[pl.ds(off + c * L, L)] for c in range(L)]
                c8 = [cols[2 * t] + cols[2 * t + 1] for t in range(8)]
                c4 = [c8[2 * t] + c8[2 * t + 1] for t in range(4)]
                c2 = [c4[0] + c4[1], c4[2] + c4[3]]
                out_v[pl.ds(i * C + g * L, L)] = c2[0] + c2[1]

        for b in range(NBUF):
            fire(b, b)

        def ring_body(kk, _):
            base = NBUF * kk
            for b in range(NBUF):
                i = base + b
                drain(i, b)
                compute(i, ubuf[b], mbuf[b], C)
                fire(i + NBUF, b)
            return 0

        n_main = (NFULL - NBUF) // NBUF
        lax.fori_loop(0, n_main, ring_body, 0)

        i0 = n_main * NBUF
        drain(i0, 0)
        compute(i0, u0, m0, C)
        ut = u0.at[pl.ds(0, TAIL)]
        mt = m0.at[pl.ds(0, TAIL)]
        tidx_u = uidx_v.at[pl.ds(NFULL * C, TAIL)]
        tidx_m = midx_v.at[pl.ds(NFULL * C, TAIL)]
        pltpu.async_copy(u_hbm.at[tidx_u], ut, su0)
        pltpu.async_copy(m_hbm.at[tidx_m], mt, sm0)
        drain(i0 + 1, 1)
        compute(i0 + 1, u1, m1, C)
        pltpu.make_async_copy(u_hbm.at[tidx_u], ut, su0).wait()
        pltpu.make_async_copy(m_hbm.at[tidx_m], mt, sm0).wait()
        compute(NFULL, u0, m0, TAIL)

        pltpu.sync_copy(out_v, out_hbm.at[pl.ds(wbase, BW)])

    return k(x_user, x_movie, u_idx, m_idx)


def kernel(x_user, x_movie, edge_label_index):
    idx = edge_label_index.astype(jnp.int32)
    return _impl(x_user, x_movie, idx[0], idx[1])
